# Initial kernel scaffold; baseline (speedup 1.0000x reference)
import jax
import jax.numpy as jnp
from jax.experimental import pallas as pl


def kernel(data, table):
    raise NotImplementedError("write your pallas kernel here")


# speedup vs baseline: 1.8317x; 1.8317x over previous
import jax
import jax.numpy as jnp
from jax import lax
from jax.experimental import pallas as pl
from jax.experimental.pallas import tpu as pltpu
from jax.experimental.pallas import tpu_sc as plsc

D_MODEL = 64
LANES = 128
G = 4


def _gather_body(table_hbm, idx_hbm, out_hbm, idx_v, rows_v, sem):
    nc = 2
    wid = lax.axis_index("s") * nc + lax.axis_index("c")
    n_chunks = idx_v.shape[0]
    base_chunk = wid * n_chunks
    pltpu.sync_copy(idx_hbm.at[pl.ds(base_chunk, n_chunks)], idx_v)

    rows_per_group = G * LANES
    n_groups = n_chunks // G

    def body(g, carry):
        copies = []
        for j in range(G):
            copies.append(pltpu.async_copy(
                table_hbm.at[idx_v.at[g * G + j]],
                rows_v.at[pl.ds(j * LANES, LANES)],
                sem))
        for c in copies:
            c.wait()
        pltpu.sync_copy(
            rows_v,
            out_hbm.at[pl.ds(base_chunk * LANES + g * rows_per_group,
                             rows_per_group)])
        return carry

    lax.fori_loop(0, n_groups, body, 0)


def kernel(data, table):
    s0, s1 = data.shape
    b = s0 * s1
    info = plsc.get_sparse_core_info()
    nw = info.num_cores * info.num_subcores
    n_chunks_total = b // LANES
    per_w = n_chunks_total // nw
    idx2d = data.reshape(n_chunks_total, LANES)

    mesh = plsc.VectorSubcoreMesh(core_axis_name="c", subcore_axis_name="s")
    out = pl.kernel(
        _gather_body,
        out_type=jax.ShapeDtypeStruct((b, D_MODEL), jnp.float32),
        mesh=mesh,
        compiler_params=pltpu.CompilerParams(use_tc_tiling_on_sc=False),
        scratch_types=[
            pltpu.VMEM((per_w, LANES), jnp.int32),
            pltpu.VMEM((G * LANES, D_MODEL), jnp.float32),
            pltpu.SemaphoreType.DMA,
        ],
    )(table, idx2d)
    return out.reshape(s0, s1, D_MODEL)


# speedup vs baseline: 1.8750x; 1.0236x over previous
import jax
import jax.numpy as jnp
from jax import lax
from jax.experimental import pallas as pl
from jax.experimental.pallas import tpu as pltpu
from jax.experimental.pallas import tpu_sc as plsc

D_MODEL = 64
LANES = 128
G = 4
NBUF = 2


def _gather_body(table_hbm, idx_hbm, out_hbm, idx_v, rows_v,
                 gsem0, gsem1, osem0, osem1):
    nc = 2
    wid = lax.axis_index("s") * nc + lax.axis_index("c")
    n_chunks = idx_v.shape[0]
    n_groups = n_chunks // G
    base_chunk = wid * n_chunks
    base_row = base_chunk * LANES
    rows_per_group = G * LANES
    gsems = (gsem0, gsem1)
    osems = (osem0, osem1)

    pltpu.sync_copy(idx_hbm.at[pl.ds(base_chunk, n_chunks)], idx_v)

    def fire_gather(g, b):
        for j in range(G):
            pltpu.async_copy(
                table_hbm.at[idx_v.at[g * G + j]],
                rows_v.at[b].at[pl.ds(j * LANES, LANES)],
                gsems[b])

    def wait_gather(b):
        for j in range(G):
            pltpu.make_async_copy(
                table_hbm.at[idx_v.at[j]],
                rows_v.at[b].at[pl.ds(j * LANES, LANES)],
                gsems[b]).wait()

    def fire_out(g, b):
        pltpu.async_copy(
            rows_v.at[b],
            out_hbm.at[pl.ds(base_row + g * rows_per_group, rows_per_group)],
            osems[b])

    def wait_out(g, b):
        pltpu.make_async_copy(
            rows_v.at[b],
            out_hbm.at[pl.ds(base_row + g * rows_per_group, rows_per_group)],
            osems[b]).wait()

    fire_gather(0, 0)
    fire_gather(1, 1)

    def body(i, carry):
        for b in range(NBUF):
            g = NBUF * i + b
            wait_gather(b)
            fire_out(g, b)
            wait_out(g, b)
            fire_gather(g + NBUF, b)
        return carry

    lax.fori_loop(0, n_groups // NBUF - 1, body, 0)
    for b in range(NBUF):
        g = n_groups - NBUF + b
        wait_gather(b)
        fire_out(g, b)
        wait_out(g, b)


def kernel(data, table):
    s0, s1 = data.shape
    b = s0 * s1
    info = plsc.get_sparse_core_info()
    nw = info.num_cores * info.num_subcores
    n_chunks_total = b // LANES
    per_w = n_chunks_total // nw
    idx2d = data.reshape(n_chunks_total, LANES)

    mesh = plsc.VectorSubcoreMesh(core_axis_name="c", subcore_axis_name="s")
    out = pl.kernel(
        _gather_body,
        out_type=jax.ShapeDtypeStruct((b, D_MODEL), jnp.float32),
        mesh=mesh,
        compiler_params=pltpu.CompilerParams(use_tc_tiling_on_sc=False),
        scratch_types=[
            pltpu.VMEM((per_w, LANES), jnp.int32),
            pltpu.VMEM((NBUF, G * LANES, D_MODEL), jnp.float32),
            pltpu.SemaphoreType.DMA,
            pltpu.SemaphoreType.DMA,
            pltpu.SemaphoreType.DMA,
            pltpu.SemaphoreType.DMA,
        ],
    )(table, idx2d)
    return out.reshape(s0, s1, D_MODEL)
